# Initial kernel scaffold; baseline (speedup 1.0000x reference)
import jax
import jax.numpy as jnp
from jax.experimental import pallas as pl


def kernel(weight, x):
    raise NotImplementedError("write your pallas kernel here")


# speedup vs baseline: 1.8429x; 1.8429x over previous
import functools

import jax
import jax.numpy as jnp
from jax import lax
from jax.experimental import pallas as pl
from jax.experimental.pallas import tpu as pltpu
from jax.experimental.pallas import tpu_sc as plsc

_info = plsc.get_sparse_core_info()
_NC, _NS = _info.num_cores, _info.num_subcores
_NW = _NC * _NS


def _emb_call(B, D, C):
    n_chunks_per_w = B // (_NW * C)
    b_per_w = B // _NW
    mesh = plsc.VectorSubcoreMesh(core_axis_name="c", subcore_axis_name="s")

    @functools.partial(
        pl.kernel,
        mesh=mesh,
        out_type=jax.ShapeDtypeStruct((B, D), jnp.float32),
        scratch_types=[
            pltpu.VMEM((C,), jnp.int32),
            pltpu.VMEM((C, D), jnp.float32),
            pltpu.SemaphoreType.DMA,
        ],
        compiler_params=pltpu.CompilerParams(use_tc_tiling_on_sc=False),
    )
    def emb(w_hbm, idx_hbm, out_hbm, idx_v, rows_v, sem):
        wid = lax.axis_index("s") * _NC + lax.axis_index("c")
        base = wid * b_per_w

        def body(g, carry):
            off = base + g * C
            pltpu.sync_copy(idx_hbm.at[pl.ds(off, C)], idx_v)
            pltpu.async_copy(w_hbm.at[idx_v], rows_v, sem).wait()
            pltpu.sync_copy(rows_v, out_hbm.at[pl.ds(off, C)])
            return carry

        lax.fori_loop(0, n_chunks_per_w, body, 0)

    return emb


def kernel(weight, x):
    B0, B1 = x.shape
    B = B0 * B1
    D = weight.shape[1]
    C = 1024
    out = _emb_call(B, D, C)(weight, x.reshape(B).astype(jnp.int32))
    return out.reshape(B0, B1, D)


# speedup vs baseline: 1.8812x; 1.0208x over previous
import functools

import jax
import jax.numpy as jnp
from jax import lax
from jax.experimental import pallas as pl
from jax.experimental.pallas import tpu as pltpu
from jax.experimental.pallas import tpu_sc as plsc

_info = plsc.get_sparse_core_info()
_NC, _NS = _info.num_cores, _info.num_subcores
_NW = _NC * _NS


def _emb_call(B, D, C, NB):
    b_per_w = B // _NW
    n_chunks = b_per_w // C
    n_groups = n_chunks // NB
    mesh = plsc.VectorSubcoreMesh(core_axis_name="c", subcore_axis_name="s")

    @functools.partial(
        pl.kernel,
        mesh=mesh,
        out_type=jax.ShapeDtypeStruct((B, D), jnp.float32),
        scratch_types=[
            pltpu.VMEM((b_per_w,), jnp.int32),
            pltpu.VMEM((NB, C, D), jnp.float32),
            [pltpu.SemaphoreType.DMA] * NB,
            [pltpu.SemaphoreType.DMA] * NB,
        ],
        compiler_params=pltpu.CompilerParams(use_tc_tiling_on_sc=False),
    )
    def emb(w_hbm, idx_hbm, out_hbm, idx_v, rows_v, gsems, ssems):
        wid = lax.axis_index("s") * _NC + lax.axis_index("c")
        base = wid * b_per_w
        pltpu.sync_copy(idx_hbm.at[pl.ds(base, b_per_w)], idx_v)

        def gather_copy(g, b):
            return pltpu.make_async_copy(
                w_hbm.at[idx_v.at[pl.ds(g * C, C)]], rows_v.at[b], gsems[b]
            )

        def scatter_copy(g, b):
            return pltpu.make_async_copy(
                rows_v.at[b], out_hbm.at[pl.ds(base + g * C, C)], ssems[b]
            )

        for b in range(NB):
            gather_copy(b, b).start()

        def body(t, carry):
            for b in range(NB):
                g = t * NB + b
                gather_copy(g, b).wait()
                scatter_copy(g, b).start()
                scatter_copy(g, b).wait()
                gather_copy(g + NB, b).start()
            return carry

        lax.fori_loop(0, n_groups - 1, body, 0)
        last = (n_groups - 1) * NB
        for b in range(NB):
            gather_copy(last + b, b).wait()
            scatter_copy(last + b, b).start()
        for b in range(NB):
            scatter_copy(last + b, b).wait()

    return emb


def kernel(weight, x):
    B0, B1 = x.shape
    B = B0 * B1
    D = weight.shape[1]
    out = _emb_call(B, D, C=512, NB=3)(weight, x.reshape(B).astype(jnp.int32))
    return out.reshape(B0, B1, D)
